# baseline (speedup 1.0000x reference)
import functools

import jax
import jax.numpy as jnp
from jax import lax
from jax.experimental import pallas as pl
from jax.experimental.pallas import tpu as pltpu
from jax.experimental.pallas import tpu_sc as plsc

_VOCAB = 1000
_VP = 1024
_LSE_PAD = 1008
_NC = 2
_NS = 16
_NW = _NC * _NS
_L = 16
_SEQ = 50
_SEQP = 56
_CH = 16
_KB = _VP // 128


def _lse_body(table_ref, out_ref):
    x = table_ref[...]
    m = jnp.max(x, axis=1)
    s = jnp.sum(jnp.exp(x - m[:, None]), axis=1)
    out_ref[...] = m + jnp.log(s)


def _compute_lse(table):
    return pl.pallas_call(
        _lse_body,
        out_shape=jax.ShapeDtypeStruct((table.shape[0],), jnp.float32),
    )(table)


def _fold_body(x_ref, o_ref):
    y = x_ref[...].reshape(_SEQP, _VP)
    o_ref[...] = y[None, :_SEQ, :_VOCAB]


def _fold(x, nbat):
    return pl.pallas_call(
        _fold_body,
        grid=(nbat,),
        in_specs=[pl.BlockSpec((_SEQP * _KB, 128), lambda i: (i, 0))],
        out_specs=pl.BlockSpec((1, _SEQ, _VOCAB), lambda i: (i, 0, 0)),
        out_shape=jax.ShapeDtypeStruct((nbat, _SEQ, _VOCAB), jnp.float32),
    )(x)


def _sc_body(table8_hbm, tflat_hbm, idx_hbm, tgt_hbm, lse_hbm,
             x_hbm, part_hbm,
             rows_v, idx_v, tgt_v, idx8_v, fidx_v, picks_v, lse_v, acc_v,
             table_sp,
             g0, g1, s0, s1, psem):
    wid = lax.axis_index("s") * _NC + lax.axis_index("c")
    batw = (x_hbm.shape[0] // (_SEQP * _KB)) // _NW
    npick = batw * _SEQP
    ib0 = wid * npick
    rb0 = wid * npick
    ncht = npick // _CH

    @pl.when(lax.axis_index("s") == 0)
    def _():
        pltpu.sync_copy(table8_hbm, table_sp)

    pltpu.sync_copy(idx_hbm.at[pl.ds(ib0, npick)], idx_v.at[pl.ds(0, npick)])
    pltpu.sync_copy(tgt_hbm.at[pl.ds(ib0, npick)], tgt_v)
    pltpu.sync_copy(lse_hbm, lse_v)
    acc_v[...] = jnp.zeros((_L,), jnp.float32)
    plsc.subcore_barrier()

    gsems = (g0, g1)
    ssems = (s0, s1)
    lane = lax.iota(jnp.int32, _L)

    def build_idx8(c, b):
        for g in range(_KB):
            p = lane + g * _L
            tok = plsc.load_gather(idx_v, [c * _CH + (p >> 3)])
            idx8_v[b, pl.ds(g * _L, _L)] = tok * _KB + (p & 7)

    def gather_desc(c, b):
        return pltpu.make_async_copy(
            table_sp.at[idx8_v.at[b]], rows_v.at[b], gsems[b])

    def scatter_desc(c, b):
        return pltpu.make_async_copy(
            rows_v.at[b],
            x_hbm.at[pl.ds((rb0 + c * _CH) * _KB, _CH * _KB)], ssems[b])

    def build_f(i, carry):
        off = i * _L
        fidx_v[pl.ds(off, _L)] = (
            idx_v[pl.ds(off, _L)] * _VP + tgt_v[pl.ds(off, _L)])
        return carry

    lax.fori_loop(0, npick // _L, build_f, None)

    def pick_desc(k):
        return pltpu.make_async_copy(
            tflat_hbm.at[fidx_v.at[pl.ds(k * 128, 128)]],
            picks_v.at[pl.ds(k * 128, 128)], psem)

    nk = npick // 128
    for k in range(nk):
        pick_desc(k).start()

    build_idx8(0, 0)
    gather_desc(0, 0).start()
    build_idx8(1, 1)
    gather_desc(1, 1).start()

    def outer(t, carry):
        for b in range(2):
            c = t * 2 + b
            gather_desc(c, b).wait()
            scatter_desc(c, b).start()

            @pl.when(c + 2 < ncht)
            def _():
                scatter_desc(c, b).wait()
                build_idx8(c + 2, b)
                gather_desc(c + 2, b).start()
        return carry

    lax.fori_loop(0, ncht // 2, outer, None)

    for k in range(nk):
        pick_desc(k).wait()

    tail_m = lax.iota(jnp.int32, _L) < (_SEQ - 3 * _L)

    def accum(r, carry):
        for g in range((_SEQ + _L - 1) // _L):
            off = r * _SEQP + g * _L
            valid = min(_L, _SEQ - g * _L)
            idxg = idx_v[pl.ds(off, _L)]
            picked = picks_v[pl.ds(off, _L)]
            if valid == _L:
                lsev = plsc.load_gather(lse_v, [idxg])
                acc_v[...] = acc_v[...] + (lsev - picked)
            else:
                lsev = plsc.load_gather(lse_v, [idxg], mask=tail_m)
                acc_v[...] = acc_v[...] + jnp.where(
                    tail_m, lsev - picked, jnp.zeros((_L,), jnp.float32))
        return carry

    lax.fori_loop(0, batw, accum, None)
    scatter_desc(ncht - 2, 0).wait()
    scatter_desc(ncht - 1, 1).wait()
    pltpu.sync_copy(acc_v, part_hbm.at[pl.ds(wid * _L, _L)])


def _sc_gather_loss(table8, tflat, idx_p, tgt_p, lse_p, nbatch):
    batw = nbatch // _NW
    call = pl.kernel(
        _sc_body,
        out_type=[
            jax.ShapeDtypeStruct((nbatch * _SEQP * _KB, 128), jnp.float32),
            jax.ShapeDtypeStruct((_NW * _L,), jnp.float32),
        ],
        mesh=plsc.VectorSubcoreMesh(core_axis_name="c", subcore_axis_name="s"),
        compiler_params=pltpu.CompilerParams(
            use_tc_tiling_on_sc=False, needs_layout_passes=False),
        scratch_types=[
            pltpu.VMEM((2, _CH * _KB, 128), jnp.float32),
            pltpu.VMEM((batw * _SEQP + _L,), jnp.int32),
            pltpu.VMEM((batw * _SEQP,), jnp.int32),
            pltpu.VMEM((2, _CH * _KB), jnp.int32),
            pltpu.VMEM((batw * _SEQP,), jnp.int32),
            pltpu.VMEM((batw * _SEQP + _L,), jnp.float32),
            pltpu.VMEM((_LSE_PAD,), jnp.float32),
            pltpu.VMEM((_L,), jnp.float32),
            pltpu.VMEM_SHARED((_VOCAB * _KB, 128), jnp.float32),
            pltpu.SemaphoreType.DMA,
            pltpu.SemaphoreType.DMA,
            pltpu.SemaphoreType.DMA,
            pltpu.SemaphoreType.DMA,
            pltpu.SemaphoreType.DMA,
        ],
    )
    return call(table8, tflat, idx_p, tgt_p, lse_p)


def kernel(idx, targets, table):
    nbat, seq = idx.shape
    lse = _compute_lse(table)
    lse_p = jnp.pad(lse, (0, _LSE_PAD - _VOCAB))
    idx_p = jnp.pad(idx, ((0, 0), (0, _SEQP - _SEQ))).reshape(-1)
    tgt_p = jnp.pad(targets, ((0, 0), (0, _SEQP - _SEQ))).reshape(-1)
    table8 = jnp.pad(table, ((0, 0), (0, _VP - _VOCAB))).reshape(-1, 128)
    tflat = jnp.pad(table, ((0, 0), (0, _VP - _VOCAB)),
                    constant_values=1.0).reshape(-1)
    x, partials = _sc_gather_loss(table8, tflat, idx_p, tgt_p, lse_p, nbat)
    logits = _fold(x, nbat)
    loss = jnp.sum(partials) / jnp.float32(nbat * seq)
    return logits, loss


# speedup vs baseline: 1.7954x; 1.7954x over previous
import functools

import jax
import jax.numpy as jnp
from jax import lax
from jax.experimental import pallas as pl
from jax.experimental.pallas import tpu as pltpu
from jax.experimental.pallas import tpu_sc as plsc

_VOCAB = 1000
_LSE_PAD = 1008
_NC = 2
_NS = 16
_NW = _NC * _NS
_L = 16
_SEQ = 50
_SEQP = 56
_PITCH = 56
_CH = 16


def _lse_body(table_ref, out_ref):
    x = table_ref[...]
    m = jnp.max(x, axis=1)
    s = jnp.sum(jnp.exp(x - m[:, None]), axis=1)
    out_ref[...] = m + jnp.log(s)


def _compute_lse(table):
    return pl.pallas_call(
        _lse_body,
        out_shape=jax.ShapeDtypeStruct((table.shape[0],), jnp.float32),
    )(table)


def _sc_body(table_hbm, tflat_hbm, idx_hbm, tgt_hbm, lse_hbm,
             rows_hbm, part_hbm,
             rows_v, idx_v, tgt_v, fidx_v, picks_v, lse_v, acc_v, table_sp,
             g0, g1, s0, s1, psem):
    wid = lax.axis_index("s") * _NC + lax.axis_index("c")
    batw = (rows_hbm.shape[0] // _SEQP) // _NW
    npick = batw * _PITCH
    ib0 = wid * npick
    rb0 = wid * batw * _SEQP
    ncht = batw * _SEQP // _CH

    @pl.when(lax.axis_index("s") == 0)
    def _():
        pltpu.sync_copy(table_hbm, table_sp)

    pltpu.sync_copy(idx_hbm.at[pl.ds(ib0, npick)], idx_v.at[pl.ds(0, npick)])
    pltpu.sync_copy(tgt_hbm.at[pl.ds(ib0, npick)], tgt_v)
    pltpu.sync_copy(lse_hbm, lse_v)
    acc_v[...] = jnp.zeros((_L,), jnp.float32)
    plsc.subcore_barrier()

    def build(i, carry):
        off = i * _L
        fidx_v[pl.ds(off, _L)] = (
            idx_v[pl.ds(off, _L)] * 1024 + tgt_v[pl.ds(off, _L)])
        return carry

    lax.fori_loop(0, npick // _L, build, None)

    def pick_desc(k):
        return pltpu.make_async_copy(
            tflat_hbm.at[fidx_v.at[pl.ds(k * 128, 128)]],
            picks_v.at[pl.ds(k * 128, 128)], psem)

    nk = npick // 128
    for k in range(nk):
        pick_desc(k).start()

    gsems = (g0, g1)
    ssems = (s0, s1)

    def gather_desc(c, b):
        return pltpu.make_async_copy(
            table_sp.at[idx_v.at[pl.ds(c * _CH, _CH)]],
            rows_v.at[b], gsems[b])

    def scatter_desc(c, b):
        return pltpu.make_async_copy(
            rows_v.at[b], rows_hbm.at[pl.ds(rb0 + c * _CH, _CH)], ssems[b])

    gather_desc(0, 0).start()
    gather_desc(1, 1).start()

    def outer(t, carry):
        for b in range(2):
            c = t * 2 + b
            gather_desc(c, b).wait()
            scatter_desc(c, b).start()

            @pl.when(c + 2 < ncht)
            def _():
                scatter_desc(c, b).wait()
                gather_desc(c + 2, b).start()
        return carry

    lax.fori_loop(0, ncht // 2, outer, None)

    for k in range(nk):
        pick_desc(k).wait()

    tail_m = lax.iota(jnp.int32, _L) < (_SEQ - 3 * _L)

    def accum(r, carry):
        for g in range((_SEQ + _L - 1) // _L):
            off = r * _PITCH + g * _L
            valid = min(_L, _SEQ - g * _L)
            if valid <= 0:
                continue
            idxg = idx_v[pl.ds(off, _L)]
            picked = picks_v[pl.ds(off, _L)]
            if valid == _L:
                lsev = plsc.load_gather(lse_v, [idxg])
                acc_v[...] = acc_v[...] + (lsev - picked)
            else:
                lsev = plsc.load_gather(lse_v, [idxg], mask=tail_m)
                acc_v[...] = acc_v[...] + jnp.where(
                    tail_m, lsev - picked, jnp.zeros((_L,), jnp.float32))
        return carry

    lax.fori_loop(0, batw, accum, None)

    scatter_desc(ncht - 2, 0).wait()
    scatter_desc(ncht - 1, 1).wait()
    pltpu.sync_copy(acc_v, part_hbm.at[pl.ds(wid * _L, _L)])


def _sc_gather_loss(table, tflat, idx_p, tgt_p, lse_p, nbatch):
    batw = nbatch // _NW
    call = pl.kernel(
        _sc_body,
        out_type=[
            jax.ShapeDtypeStruct((nbatch * _SEQP, _VOCAB), jnp.float32),
            jax.ShapeDtypeStruct((_NW * _L,), jnp.float32),
        ],
        mesh=plsc.VectorSubcoreMesh(core_axis_name="c", subcore_axis_name="s"),
        compiler_params=pltpu.CompilerParams(
            use_tc_tiling_on_sc=False, needs_layout_passes=False),
        scratch_types=[
            pltpu.VMEM((2, _CH, _VOCAB), jnp.float32),
            pltpu.VMEM((batw * _PITCH + _L,), jnp.int32),
            pltpu.VMEM((batw * _PITCH,), jnp.int32),
            pltpu.VMEM((batw * _PITCH,), jnp.int32),
            pltpu.VMEM((batw * _PITCH + _L,), jnp.float32),
            pltpu.VMEM((_LSE_PAD,), jnp.float32),
            pltpu.VMEM((_L,), jnp.float32),
            pltpu.VMEM_SHARED((_VOCAB, _VOCAB), jnp.float32),
            pltpu.SemaphoreType.DMA,
            pltpu.SemaphoreType.DMA,
            pltpu.SemaphoreType.DMA,
            pltpu.SemaphoreType.DMA,
            pltpu.SemaphoreType.DMA,
        ],
    )
    return call(table, tflat, idx_p, tgt_p, lse_p)


def kernel(idx, targets, table):
    nbat, seq = idx.shape
    lse = _compute_lse(table)
    lse_p = jnp.pad(lse, (0, _LSE_PAD - _VOCAB))
    idx_p = jnp.pad(idx, ((0, 0), (0, _PITCH - _SEQ))).reshape(-1)
    tgt_p = jnp.pad(targets, ((0, 0), (0, _PITCH - _SEQ))).reshape(-1)
    tflat = jnp.pad(table, ((0, 0), (0, 1024 - _VOCAB))).reshape(-1)
    rows, partials = _sc_gather_loss(table, tflat, idx_p, tgt_p, lse_p, nbat)
    logits = rows.reshape(nbat, _SEQP, _VOCAB)[:, :_SEQ, :]
    loss = jnp.sum(partials) / jnp.float32(nbat * seq)
    return logits, loss
